# Initial kernel scaffold; baseline (speedup 1.0000x reference)
import jax
import jax.numpy as jnp
from jax.experimental import pallas as pl


def kernel(x, edge_index, W0, b0, W1, b1, W2, b2, W3, b3, W4, b4):
    raise NotImplementedError("write your pallas kernel here")


# speedup vs baseline: 16.8017x; 16.8017x over previous
import jax
import jax.numpy as jnp
from jax import lax
from jax.experimental import pallas as pl
from jax.experimental.pallas import tpu as pltpu
from jax.experimental.pallas import tpu_sc as plsc

_N = 10000
_E = 320000
_NC = 2
_NS = 16
_NW = _NC * _NS
_CH = 80
_NROW = _E // _CH
_KJS = _NROW // _NS
_KJW = _NROW // _NW
_NP = 10240
_RPT = _NP // _NS
_DH = 64
_RB = 400

_mesh = plsc.VectorSubcoreMesh(core_axis_name="c", subcore_axis_name="s")
_sc_params = pltpu.CompilerParams(use_tc_tiling_on_sc=False)


def _seg_sum_split(hpp3, row3, col3, zeros):
  @pl.kernel(
      out_type=jax.ShapeDtypeStruct((_NC, _NP, _DH), jnp.float32),
      mesh=_mesh,
      compiler_params=_sc_params,
      scratch_types=[
          pltpu.VMEM((_KJS, _CH), jnp.int32),
          pltpu.VMEM((_KJS, _CH), jnp.int32),
          pltpu.VMEM((_CH, _DH), jnp.float32),
          pltpu.VMEM((_CH, _DH), jnp.float32),
          pltpu.SemaphoreType.DMA,
          pltpu.SemaphoreType.DMA,
          pltpu.VMEM_SHARED((_NP, _DH), jnp.float32),
      ],
  )
  def k(hpp_hbm, row_hbm, col_hbm, zeros_hbm, out_hbm,
        row_v, col_v, msg0, msg1, sem0, sem1, acc_sh):
    cid = lax.axis_index("c")
    sid = lax.axis_index("s")
    tbl = hpp_hbm.at[cid]
    pltpu.sync_copy(zeros_hbm, acc_sh.at[pl.ds(sid * _RPT, _RPT)])
    pltpu.sync_copy(row_hbm.at[sid], row_v)
    pltpu.sync_copy(col_hbm.at[sid], col_v)
    plsc.subcore_barrier()

    pltpu.async_copy(tbl.at[row_v.at[0]], msg0, sem0)

    @pl.loop(0, _KJS - 2, step=2)
    def _(j):
      pltpu.async_copy(tbl.at[row_v.at[j + 1]], msg1, sem1)
      pltpu.make_async_copy(tbl.at[row_v.at[j]], msg0, sem0).wait()
      pltpu.sync_copy(msg0, acc_sh.at[col_v.at[j]], add=True)
      pltpu.async_copy(tbl.at[row_v.at[j + 2]], msg0, sem0)
      pltpu.make_async_copy(tbl.at[row_v.at[j + 1]], msg1, sem1).wait()
      pltpu.sync_copy(msg1, acc_sh.at[col_v.at[j + 1]], add=True)

    pltpu.async_copy(tbl.at[row_v.at[_KJS - 1]], msg1, sem1)
    pltpu.make_async_copy(tbl.at[row_v.at[_KJS - 2]], msg0, sem0).wait()
    pltpu.sync_copy(msg0, acc_sh.at[col_v.at[_KJS - 2]], add=True)
    pltpu.make_async_copy(tbl.at[row_v.at[_KJS - 1]], msg1, sem1).wait()
    pltpu.sync_copy(msg1, acc_sh.at[col_v.at[_KJS - 1]], add=True)

    plsc.subcore_barrier()
    pltpu.sync_copy(acc_sh.at[pl.ds(sid * _RPT, _RPT)],
                    out_hbm.at[cid, pl.ds(sid * _RPT, _RPT)])

  return k(hpp3, row3, col3, zeros)


def _seg_sum_part(hpp, row2, col2, zeros, D):
  @pl.kernel(
      out_type=jax.ShapeDtypeStruct((_NC, _NP, D), jnp.float32),
      mesh=_mesh,
      compiler_params=_sc_params,
      scratch_types=[
          pltpu.VMEM((_KJW, _CH), jnp.int32),
          pltpu.VMEM((_KJW, _CH), jnp.int32),
          pltpu.VMEM((_CH, D), jnp.float32),
          pltpu.VMEM((_CH, D), jnp.float32),
          pltpu.SemaphoreType.DMA,
          pltpu.SemaphoreType.DMA,
          pltpu.VMEM_SHARED((_NP, D), jnp.float32),
      ],
  )
  def k(hpp_hbm, row_hbm, col_hbm, zeros_hbm, out_hbm,
        row_v, col_v, msg0, msg1, sem0, sem1, acc_sh):
    cid = lax.axis_index("c")
    sid = lax.axis_index("s")
    wid = sid * _NC + cid
    pltpu.sync_copy(zeros_hbm, acc_sh.at[pl.ds(sid * _RPT, _RPT)])
    pltpu.sync_copy(row_hbm.at[wid], row_v)
    pltpu.sync_copy(col_hbm.at[wid], col_v)
    plsc.subcore_barrier()

    pltpu.async_copy(hpp_hbm.at[row_v.at[0]], msg0, sem0)

    @pl.loop(0, _KJW - 1, step=2)
    def _(j):
      pltpu.async_copy(hpp_hbm.at[row_v.at[j + 1]], msg1, sem1)
      pltpu.make_async_copy(hpp_hbm.at[row_v.at[j]], msg0, sem0).wait()
      pltpu.sync_copy(msg0, acc_sh.at[col_v.at[j]], add=True)
      pltpu.async_copy(hpp_hbm.at[row_v.at[j + 2]], msg0, sem0)
      pltpu.make_async_copy(hpp_hbm.at[row_v.at[j + 1]], msg1, sem1).wait()
      pltpu.sync_copy(msg1, acc_sh.at[col_v.at[j + 1]], add=True)

    pltpu.make_async_copy(hpp_hbm.at[row_v.at[_KJW - 1]], msg0, sem0).wait()
    pltpu.sync_copy(msg0, acc_sh.at[col_v.at[_KJW - 1]], add=True)
    plsc.subcore_barrier()
    pltpu.sync_copy(acc_sh.at[pl.ds(sid * _RPT, _RPT)],
                    out_hbm.at[cid, pl.ds(sid * _RPT, _RPT)])

  return k(hpp, row2, col2, zeros)


def _deg_hist(col2, ones, zeros):
  @pl.kernel(
      out_type=jax.ShapeDtypeStruct((_NC, _NP, 16), jnp.float32),
      mesh=_mesh,
      compiler_params=_sc_params,
      scratch_types=[
          pltpu.VMEM((_KJW, _CH), jnp.int32),
          pltpu.VMEM((_CH, 16), jnp.float32),
          pltpu.VMEM_SHARED((_NP, 16), jnp.float32),
      ],
  )
  def k(col_hbm, ones_hbm, zeros_hbm, out_hbm, col_v, ones_v, acc_sh):
    cid = lax.axis_index("c")
    sid = lax.axis_index("s")
    wid = sid * _NC + cid
    pltpu.sync_copy(zeros_hbm, acc_sh.at[pl.ds(sid * _RPT, _RPT)])
    pltpu.sync_copy(col_hbm.at[wid], col_v)
    pltpu.sync_copy(ones_hbm, ones_v)
    plsc.subcore_barrier()

    @pl.loop(0, _KJW)
    def _(j):
      pltpu.sync_copy(ones_v, acc_sh.at[col_v.at[j]], add=True)

    plsc.subcore_barrier()
    pltpu.sync_copy(acc_sh.at[pl.ds(sid * _RPT, _RPT)],
                    out_hbm.at[cid, pl.ds(sid * _RPT, _RPT)])

  return k(col2, ones, zeros)


def _g_from_deg(dacc):
  def body(d_ref, o_ref):
    deg = d_ref[0, :, :1] + d_ref[1, :, :1] + 1.0
    o_ref[...] = lax.rsqrt(deg)

  return pl.pallas_call(
      body,
      grid=(_N // _RB,),
      in_specs=[pl.BlockSpec((_NC, _RB, 16), lambda i: (0, i, 0))],
      out_specs=pl.BlockSpec((_RB, 1), lambda i: (i, 0)),
      out_shape=jax.ShapeDtypeStruct((_N, 1), jnp.float32),
  )(dacc)


def _mm_scale(x, W, g):
  def body(x_ref, w_ref, g_ref, o_ref):
    h = lax.dot_general(x_ref[...], w_ref[...], (((1,), (0,)), ((), ())),
                        preferred_element_type=jnp.float32,
                        precision=lax.Precision.HIGHEST)
    h = h * g_ref[...]
    o_ref[0] = h[:, :_DH]
    o_ref[1] = h[:, _DH:]

  K = W.shape[0]
  return pl.pallas_call(
      body,
      grid=(_N // _RB,),
      in_specs=[pl.BlockSpec((_RB, K), lambda i: (i, 0)),
                pl.BlockSpec((K, 2 * _DH), lambda i: (0, 0)),
                pl.BlockSpec((_RB, 1), lambda i: (i, 0))],
      out_specs=pl.BlockSpec((_NC, _RB, _DH), lambda i: (0, i, 0)),
      out_shape=jax.ShapeDtypeStruct((_NC, _N, _DH), jnp.float32),
  )(x, W, g)


def _mid(acc, hpp3, g, b, W, split_out):
  def body(a_ref, h_ref, g_ref, b_ref, w_ref, o_ref):
    s0 = a_ref[0] + h_ref[0]
    s1 = a_ref[1] + h_ref[1]
    s = jnp.concatenate([s0, s1], axis=1)
    t = jnp.maximum(g_ref[...] * s + b_ref[...], 0.0)
    h = lax.dot_general(t, w_ref[...], (((1,), (0,)), ((), ())),
                        preferred_element_type=jnp.float32,
                        precision=lax.Precision.HIGHEST)
    h = h * g_ref[...]
    if split_out:
      o_ref[0] = h[:, :_DH]
      o_ref[1] = h[:, _DH:]
    else:
      o_ref[...] = h

  K, Dout = W.shape
  if split_out:
    out_spec = pl.BlockSpec((_NC, _RB, _DH), lambda i: (0, i, 0))
    out_shape = jax.ShapeDtypeStruct((_NC, _N, _DH), jnp.float32)
  else:
    out_spec = pl.BlockSpec((_RB, Dout), lambda i: (i, 0))
    out_shape = jax.ShapeDtypeStruct((_N, Dout), jnp.float32)
  return pl.pallas_call(
      body,
      grid=(_N // _RB,),
      in_specs=[pl.BlockSpec((_NC, _RB, _DH), lambda i: (0, i, 0)),
                pl.BlockSpec((_NC, _RB, _DH), lambda i: (0, i, 0)),
                pl.BlockSpec((_RB, 1), lambda i: (i, 0)),
                pl.BlockSpec((1, K), lambda i: (0, 0)),
                pl.BlockSpec((K, Dout), lambda i: (0, 0))],
      out_specs=out_spec,
      out_shape=out_shape,
  )(acc, hpp3, g, b, W)


def _final(acc, hpp, g, b):
  def body(a_ref, h_ref, g_ref, b_ref, o_ref):
    o_ref[...] = g_ref[...] * (a_ref[0] + a_ref[1] + h_ref[...]) + b_ref[...]

  D = hpp.shape[1]
  return pl.pallas_call(
      body,
      grid=(_N // _RB,),
      in_specs=[pl.BlockSpec((_NC, _RB, D), lambda i: (0, i, 0)),
                pl.BlockSpec((_RB, D), lambda i: (i, 0)),
                pl.BlockSpec((_RB, 1), lambda i: (i, 0)),
                pl.BlockSpec((1, D), lambda i: (0, 0))],
      out_specs=pl.BlockSpec((_RB, D), lambda i: (i, 0)),
      out_shape=jax.ShapeDtypeStruct((_N, D), jnp.float32),
  )(acc, hpp, g, b)


def kernel(x, edge_index, W0, b0, W1, b1, W2, b2, W3, b3, W4, b4):
  row_w = edge_index[0].reshape(_NW, _KJW, _CH)
  col_w = edge_index[1].reshape(_NW, _KJW, _CH)
  row_s = edge_index[0].reshape(_NS, _KJS, _CH)
  col_s = edge_index[1].reshape(_NS, _KJS, _CH)
  zeros64 = jnp.zeros((_RPT, _DH), jnp.float32)
  zeros16 = jnp.zeros((_RPT, 16), jnp.float32)
  ones16 = jnp.ones((_CH, 16), jnp.float32)

  dacc = _deg_hist(col_w, ones16, zeros16)
  g = _g_from_deg(dacc)

  W4p = jnp.pad(W4, ((0, 0), (0, 14)))
  b4p = jnp.pad(b4, (0, 14)).reshape(1, 16)

  hpp3 = _mm_scale(x, W0, g)
  bs = (b0.reshape(1, -1), b1.reshape(1, -1), b2.reshape(1, -1),
        b3.reshape(1, -1))
  Ws = (W1, W2, W3, W4p)
  for i in range(4):
    acc = _seg_sum_split(hpp3, row_s, col_s, zeros64)
    hpp3 = _mid(acc, hpp3, g, bs[i], Ws[i], split_out=(i < 3))
  acc = _seg_sum_part(hpp3, row_w, col_w, zeros16, 16)
  out16 = _final(acc, hpp3, g, b4p)
  return out16[:, :2]


# speedup vs baseline: 21.9865x; 1.3086x over previous
import jax
import jax.numpy as jnp
from jax import lax
from jax.experimental import pallas as pl
from jax.experimental.pallas import tpu as pltpu
from jax.experimental.pallas import tpu_sc as plsc

_N = 10000
_E = 320000
_NC = 2
_NS = 16
_NW = _NC * _NS
_CH = 80
_KJS = _E // _NS // _CH
_KJW = _E // _NW // _CH
_NP = 10240
_RPT = _NP // _NS
_DH = 64
_NB = 8
_RB = 400

_mesh = plsc.VectorSubcoreMesh(core_axis_name="c", subcore_axis_name="s")
_sc_params = pltpu.CompilerParams(use_tc_tiling_on_sc=False)


def _ring_pipeline(tbl, row_v, col_v, acc_sh, msgs, gs, ss, kj):
  def wait_gather(jj, b):
    pltpu.make_async_copy(tbl.at[row_v.at[jj]], msgs[b], gs[b]).wait()

  def start_scatter(jj, b):
    pltpu.async_copy(msgs[b], acc_sh.at[col_v.at[jj]], ss[b], add=True)

  def drain_scatter(b):
    pltpu.make_async_copy(msgs[b], acc_sh.at[col_v.at[0]], ss[b]).wait()

  for jj in range(4):
    pltpu.async_copy(tbl.at[row_v.at[jj]], msgs[jj], gs[jj])
  for jj in range(4):
    wait_gather(jj, jj)
    start_scatter(jj, jj)
    pltpu.async_copy(tbl.at[row_v.at[jj + 4]], msgs[jj + 4], gs[jj + 4])

  main_hi = 4 + 8 * ((kj - 8) // 8)

  @pl.loop(4, main_hi, step=8)
  def _(jj0):
    for u in range(8):
      jj = jj0 + u
      b = (4 + u) % _NB
      wait_gather(jj, b)
      start_scatter(jj, b)
      b4 = (4 + u + 4) % _NB
      drain_scatter(b4)
      pltpu.async_copy(tbl.at[row_v.at[jj + 4]], msgs[b4], gs[b4])

  for jj in range(main_hi, kj):
    b = jj % _NB
    wait_gather(jj, b)
    start_scatter(jj, b)
    if jj + 4 < kj:
      b4 = (jj + 4) % _NB
      drain_scatter(b4)
      pltpu.async_copy(tbl.at[row_v.at[jj + 4]], msgs[b4], gs[b4])
  for b in range(_NB):
    drain_scatter(b)


def _sc_scratch(kj, d):
  return ([pltpu.VMEM((kj, _CH), jnp.int32),
           pltpu.VMEM((kj, _CH), jnp.int32)]
          + [pltpu.VMEM((_CH, d), jnp.float32) for _ in range(_NB)]
          + [pltpu.SemaphoreType.DMA for _ in range(2 * _NB)])


def _seg_sum_split(hpp3, row3, col3, zeros):
  @pl.kernel(
      out_type=jax.ShapeDtypeStruct((_NC, _NP, _DH), jnp.float32),
      mesh=_mesh,
      compiler_params=_sc_params,
      scratch_types=_sc_scratch(_KJS, _DH)
      + [pltpu.VMEM_SHARED((_NP, _DH), jnp.float32)],
  )
  def k(hpp_hbm, row_hbm, col_hbm, zeros_hbm, out_hbm,
        row_v, col_v, *rest):
    msgs, gs, ss, acc_sh = (rest[:_NB], rest[_NB:2 * _NB],
                            rest[2 * _NB:3 * _NB], rest[3 * _NB])
    cid = lax.axis_index("c")
    sid = lax.axis_index("s")
    tbl = hpp_hbm.at[cid]
    pltpu.sync_copy(zeros_hbm, acc_sh.at[pl.ds(sid * _RPT, _RPT)])
    pltpu.sync_copy(row_hbm.at[sid], row_v)
    pltpu.sync_copy(col_hbm.at[sid], col_v)
    plsc.subcore_barrier()
    _ring_pipeline(tbl, row_v, col_v, acc_sh, msgs, gs, ss, _KJS)
    plsc.subcore_barrier()
    pltpu.sync_copy(acc_sh.at[pl.ds(sid * _RPT, _RPT)],
                    out_hbm.at[cid, pl.ds(sid * _RPT, _RPT)])

  return k(hpp3, row3, col3, zeros)


def _seg_sum_part(hpp, row2, col2, zeros, D):
  @pl.kernel(
      out_type=jax.ShapeDtypeStruct((_NC, _NP, D), jnp.float32),
      mesh=_mesh,
      compiler_params=_sc_params,
      scratch_types=_sc_scratch(_KJW, D)
      + [pltpu.VMEM_SHARED((_NP, D), jnp.float32)],
  )
  def k(hpp_hbm, row_hbm, col_hbm, zeros_hbm, out_hbm,
        row_v, col_v, *rest):
    msgs, gs, ss, acc_sh = (rest[:_NB], rest[_NB:2 * _NB],
                            rest[2 * _NB:3 * _NB], rest[3 * _NB])
    cid = lax.axis_index("c")
    sid = lax.axis_index("s")
    wid = sid * _NC + cid
    pltpu.sync_copy(zeros_hbm, acc_sh.at[pl.ds(sid * _RPT, _RPT)])
    pltpu.sync_copy(row_hbm.at[wid], row_v)
    pltpu.sync_copy(col_hbm.at[wid], col_v)
    plsc.subcore_barrier()
    _ring_pipeline(hpp_hbm, row_v, col_v, acc_sh, msgs, gs, ss, _KJW)
    plsc.subcore_barrier()
    pltpu.sync_copy(acc_sh.at[pl.ds(sid * _RPT, _RPT)],
                    out_hbm.at[cid, pl.ds(sid * _RPT, _RPT)])

  return k(hpp, row2, col2, zeros)


def _deg_hist(col2, ones, zeros):
  @pl.kernel(
      out_type=jax.ShapeDtypeStruct((_NC, _NP, 16), jnp.float32),
      mesh=_mesh,
      compiler_params=_sc_params,
      scratch_types=[
          pltpu.VMEM((_KJW, _CH), jnp.int32),
          pltpu.VMEM((_CH, 16), jnp.float32),
          pltpu.SemaphoreType.DMA,
          pltpu.VMEM_SHARED((_NP, 16), jnp.float32),
      ],
  )
  def k(col_hbm, ones_hbm, zeros_hbm, out_hbm, col_v, ones_v, sem, acc_sh):
    cid = lax.axis_index("c")
    sid = lax.axis_index("s")
    wid = sid * _NC + cid
    pltpu.sync_copy(zeros_hbm, acc_sh.at[pl.ds(sid * _RPT, _RPT)])
    pltpu.sync_copy(col_hbm.at[wid], col_v)
    pltpu.sync_copy(ones_hbm, ones_v)
    plsc.subcore_barrier()

    @pl.loop(0, _KJW)
    def _(j):
      pltpu.async_copy(ones_v, acc_sh.at[col_v.at[j]], sem, add=True)

    @pl.loop(0, _KJW)
    def _(j):
      pltpu.make_async_copy(ones_v, acc_sh.at[col_v.at[0]], sem).wait()

    plsc.subcore_barrier()
    pltpu.sync_copy(acc_sh.at[pl.ds(sid * _RPT, _RPT)],
                    out_hbm.at[cid, pl.ds(sid * _RPT, _RPT)])

  return k(col2, ones, zeros)


def _mm_plain(x, W):
  def body(x_ref, w_ref, o_ref):
    o_ref[...] = lax.dot_general(x_ref[...], w_ref[...],
                                 (((1,), (0,)), ((), ())),
                                 preferred_element_type=jnp.float32,
                                 precision=lax.Precision.HIGHEST)

  K, Dout = W.shape
  return pl.pallas_call(
      body,
      grid=(_N // _RB,),
      in_specs=[pl.BlockSpec((_RB, K), lambda i: (i, 0)),
                pl.BlockSpec((K, Dout), lambda i: (0, 0))],
      out_specs=pl.BlockSpec((_RB, Dout), lambda i: (i, 0)),
      out_shape=jax.ShapeDtypeStruct((_N, Dout), jnp.float32),
  )(x, W)


def _g_scale(dacc, h0):
  def body(d_ref, h_ref, g_ref, o_ref):
    deg = d_ref[0, :, :1] + d_ref[1, :, :1] + 1.0
    g = lax.rsqrt(deg)
    g_ref[...] = g
    h = h_ref[...] * g
    o_ref[0] = h[:, :_DH]
    o_ref[1] = h[:, _DH:]

  return pl.pallas_call(
      body,
      grid=(_N // _RB,),
      in_specs=[pl.BlockSpec((_NC, _RB, 16), lambda i: (0, i, 0)),
                pl.BlockSpec((_RB, 2 * _DH), lambda i: (i, 0))],
      out_specs=[pl.BlockSpec((_RB, 1), lambda i: (i, 0)),
                 pl.BlockSpec((_NC, _RB, _DH), lambda i: (0, i, 0))],
      out_shape=[jax.ShapeDtypeStruct((_N, 1), jnp.float32),
                 jax.ShapeDtypeStruct((_NC, _N, _DH), jnp.float32)],
  )(dacc, h0)


def _mid(acc, hpp3, g, b, W, split_out):
  def body(a_ref, h_ref, g_ref, b_ref, w_ref, o_ref):
    s0 = a_ref[0] + h_ref[0]
    s1 = a_ref[1] + h_ref[1]
    s = jnp.concatenate([s0, s1], axis=1)
    t = jnp.maximum(g_ref[...] * s + b_ref[...], 0.0)
    h = lax.dot_general(t, w_ref[...], (((1,), (0,)), ((), ())),
                        preferred_element_type=jnp.float32,
                        precision=lax.Precision.HIGHEST)
    h = h * g_ref[...]
    if split_out:
      o_ref[0] = h[:, :_DH]
      o_ref[1] = h[:, _DH:]
    else:
      o_ref[...] = h

  K, Dout = W.shape
  if split_out:
    out_spec = pl.BlockSpec((_NC, _RB, _DH), lambda i: (0, i, 0))
    out_shape = jax.ShapeDtypeStruct((_NC, _N, _DH), jnp.float32)
  else:
    out_spec = pl.BlockSpec((_RB, Dout), lambda i: (i, 0))
    out_shape = jax.ShapeDtypeStruct((_N, Dout), jnp.float32)
  return pl.pallas_call(
      body,
      grid=(_N // _RB,),
      in_specs=[pl.BlockSpec((_NC, _RB, _DH), lambda i: (0, i, 0)),
                pl.BlockSpec((_NC, _RB, _DH), lambda i: (0, i, 0)),
                pl.BlockSpec((_RB, 1), lambda i: (i, 0)),
                pl.BlockSpec((1, K), lambda i: (0, 0)),
                pl.BlockSpec((K, Dout), lambda i: (0, 0))],
      out_specs=out_spec,
      out_shape=out_shape,
  )(acc, hpp3, g, b, W)


def _final(acc, hpp, g, b):
  def body(a_ref, h_ref, g_ref, b_ref, o_ref):
    o_ref[...] = g_ref[...] * (a_ref[0] + a_ref[1] + h_ref[...]) + b_ref[...]

  D = hpp.shape[1]
  return pl.pallas_call(
      body,
      grid=(_N // _RB,),
      in_specs=[pl.BlockSpec((_NC, _RB, D), lambda i: (0, i, 0)),
                pl.BlockSpec((_RB, D), lambda i: (i, 0)),
                pl.BlockSpec((_RB, 1), lambda i: (i, 0)),
                pl.BlockSpec((1, D), lambda i: (0, 0))],
      out_specs=pl.BlockSpec((_RB, D), lambda i: (i, 0)),
      out_shape=jax.ShapeDtypeStruct((_N, D), jnp.float32),
  )(acc, hpp, g, b)


def kernel(x, edge_index, W0, b0, W1, b1, W2, b2, W3, b3, W4, b4):
  row_w = edge_index[0].reshape(_NW, _KJW, _CH)
  col_w = edge_index[1].reshape(_NW, _KJW, _CH)
  row_s = edge_index[0].reshape(_NS, _KJS, _CH)
  col_s = edge_index[1].reshape(_NS, _KJS, _CH)
  zeros64 = jnp.zeros((_RPT, _DH), jnp.float32)
  zeros16 = jnp.zeros((_RPT, 16), jnp.float32)
  ones16 = jnp.ones((_CH, 16), jnp.float32)

  dacc = _deg_hist(col_w, ones16, zeros16)
  h0 = _mm_plain(x, W0)
  g, hpp3 = _g_scale(dacc, h0)

  W4p = jnp.pad(W4, ((0, 0), (0, 14)))
  b4p = jnp.pad(b4, (0, 14)).reshape(1, 16)

  bs = (b0.reshape(1, -1), b1.reshape(1, -1), b2.reshape(1, -1),
        b3.reshape(1, -1))
  Ws = (W1, W2, W3, W4p)
  for i in range(4):
    acc = _seg_sum_split(hpp3, row_s, col_s, zeros64)
    hpp3 = _mid(acc, hpp3, g, bs[i], Ws[i], split_out=(i < 3))
  acc = _seg_sum_part(hpp3, row_w, col_w, zeros16, 16)
  out16 = _final(acc, hpp3, g, b4p)
  return out16[:, :2]


# speedup vs baseline: 24.0409x; 1.0934x over previous
import jax
import jax.numpy as jnp
from jax import lax
from jax.experimental import pallas as pl
from jax.experimental.pallas import tpu as pltpu
from jax.experimental.pallas import tpu_sc as plsc

_N = 10000
_E = 320000
_NC = 2
_NS = 16
_NW = _NC * _NS
_CH = 80
_KJS = _E // _NS // _CH
_KJW = _E // _NW // _CH
_NP = 10240
_RPT = _NP // _NS
_DH = 64
_NB = 8
_RB = 2000

_mesh = plsc.VectorSubcoreMesh(core_axis_name="c", subcore_axis_name="s")
_sc_params = pltpu.CompilerParams(use_tc_tiling_on_sc=False)


def _ring_pipeline(tbl, row_v, col_v, acc_sh, msgs, gs, ss, kj):
  def wait_gather(jj, b):
    pltpu.make_async_copy(tbl.at[row_v.at[jj]], msgs[b], gs[b]).wait()

  def start_scatter(jj, b):
    pltpu.async_copy(msgs[b], acc_sh.at[col_v.at[jj]], ss[b], add=True)

  def drain_scatter(b):
    pltpu.make_async_copy(msgs[b], acc_sh.at[col_v.at[0]], ss[b]).wait()

  for jj in range(4):
    pltpu.async_copy(tbl.at[row_v.at[jj]], msgs[jj], gs[jj])
  for jj in range(4):
    wait_gather(jj, jj)
    start_scatter(jj, jj)
    pltpu.async_copy(tbl.at[row_v.at[jj + 4]], msgs[jj + 4], gs[jj + 4])

  main_hi = 4 + 8 * ((kj - 8) // 8)

  @pl.loop(4, main_hi, step=8)
  def _(jj0):
    for u in range(8):
      jj = jj0 + u
      b = (4 + u) % _NB
      wait_gather(jj, b)
      start_scatter(jj, b)
      b4 = (4 + u + 4) % _NB
      drain_scatter(b4)
      pltpu.async_copy(tbl.at[row_v.at[jj + 4]], msgs[b4], gs[b4])

  for jj in range(main_hi, kj):
    b = jj % _NB
    wait_gather(jj, b)
    start_scatter(jj, b)
    if jj + 4 < kj:
      b4 = (jj + 4) % _NB
      drain_scatter(b4)
      pltpu.async_copy(tbl.at[row_v.at[jj + 4]], msgs[b4], gs[b4])
  for b in range(_NB):
    drain_scatter(b)


def _sc_scratch(kj, d):
  return ([pltpu.VMEM((kj, _CH), jnp.int32),
           pltpu.VMEM((kj, _CH), jnp.int32)]
          + [pltpu.VMEM((_CH, d), jnp.float32) for _ in range(_NB)]
          + [pltpu.SemaphoreType.DMA for _ in range(2 * _NB)])


def _seg_sum_split(hpp3, row3, col3, zeros):
  @pl.kernel(
      out_type=jax.ShapeDtypeStruct((_NC, _NP, _DH), jnp.float32),
      mesh=_mesh,
      compiler_params=_sc_params,
      scratch_types=_sc_scratch(_KJS, _DH)
      + [pltpu.VMEM_SHARED((_NP, _DH), jnp.float32)],
  )
  def k(hpp_hbm, row_hbm, col_hbm, zeros_hbm, out_hbm,
        row_v, col_v, *rest):
    msgs, gs, ss, acc_sh = (rest[:_NB], rest[_NB:2 * _NB],
                            rest[2 * _NB:3 * _NB], rest[3 * _NB])
    cid = lax.axis_index("c")
    sid = lax.axis_index("s")
    tbl = hpp_hbm.at[cid]
    pltpu.sync_copy(zeros_hbm, acc_sh.at[pl.ds(sid * _RPT, _RPT)])
    pltpu.sync_copy(row_hbm.at[2 * sid], row_v.at[pl.ds(0, _KJW)])
    pltpu.sync_copy(row_hbm.at[2 * sid + 1], row_v.at[pl.ds(_KJW, _KJW)])
    pltpu.sync_copy(col_hbm.at[2 * sid], col_v.at[pl.ds(0, _KJW)])
    pltpu.sync_copy(col_hbm.at[2 * sid + 1], col_v.at[pl.ds(_KJW, _KJW)])
    plsc.subcore_barrier()
    _ring_pipeline(tbl, row_v, col_v, acc_sh, msgs, gs, ss, _KJS)
    plsc.subcore_barrier()
    pltpu.sync_copy(acc_sh.at[pl.ds(sid * _RPT, _RPT)],
                    out_hbm.at[cid, pl.ds(sid * _RPT, _RPT)])

  return k(hpp3, row3, col3, zeros)


def _seg_sum_part(hpp, row2, col2, zeros, D):
  @pl.kernel(
      out_type=jax.ShapeDtypeStruct((_NC, _NP, D), jnp.float32),
      mesh=_mesh,
      compiler_params=_sc_params,
      scratch_types=_sc_scratch(_KJW, D)
      + [pltpu.VMEM_SHARED((_NP, D), jnp.float32)],
  )
  def k(hpp_hbm, row_hbm, col_hbm, zeros_hbm, out_hbm,
        row_v, col_v, *rest):
    msgs, gs, ss, acc_sh = (rest[:_NB], rest[_NB:2 * _NB],
                            rest[2 * _NB:3 * _NB], rest[3 * _NB])
    cid = lax.axis_index("c")
    sid = lax.axis_index("s")
    wid = sid * _NC + cid
    pltpu.sync_copy(zeros_hbm, acc_sh.at[pl.ds(sid * _RPT, _RPT)])
    pltpu.sync_copy(row_hbm.at[wid], row_v)
    pltpu.sync_copy(col_hbm.at[wid], col_v)
    plsc.subcore_barrier()
    _ring_pipeline(hpp_hbm, row_v, col_v, acc_sh, msgs, gs, ss, _KJW)
    plsc.subcore_barrier()
    pltpu.sync_copy(acc_sh.at[pl.ds(sid * _RPT, _RPT)],
                    out_hbm.at[cid, pl.ds(sid * _RPT, _RPT)])

  return k(hpp, row2, col2, zeros)


def _deg_hist(col2, ones, zeros):
  @pl.kernel(
      out_type=jax.ShapeDtypeStruct((_NC, _NP, 16), jnp.float32),
      mesh=_mesh,
      compiler_params=_sc_params,
      scratch_types=[
          pltpu.VMEM((_KJW, _CH), jnp.int32),
          pltpu.VMEM((_CH, 16), jnp.float32),
          pltpu.SemaphoreType.DMA,
          pltpu.VMEM_SHARED((_NP, 16), jnp.float32),
      ],
  )
  def k(col_hbm, ones_hbm, zeros_hbm, out_hbm, col_v, ones_v, sem, acc_sh):
    cid = lax.axis_index("c")
    sid = lax.axis_index("s")
    wid = sid * _NC + cid
    pltpu.sync_copy(zeros_hbm, acc_sh.at[pl.ds(sid * _RPT, _RPT)])
    pltpu.sync_copy(col_hbm.at[wid], col_v)
    pltpu.sync_copy(ones_hbm, ones_v)
    plsc.subcore_barrier()

    @pl.loop(0, _KJW)
    def _(j):
      pltpu.async_copy(ones_v, acc_sh.at[col_v.at[j]], sem, add=True)

    @pl.loop(0, _KJW)
    def _(j):
      pltpu.make_async_copy(ones_v, acc_sh.at[col_v.at[0]], sem).wait()

    plsc.subcore_barrier()
    pltpu.sync_copy(acc_sh.at[pl.ds(sid * _RPT, _RPT)],
                    out_hbm.at[cid, pl.ds(sid * _RPT, _RPT)])

  return k(col2, ones, zeros)


def _mm_plain(x, W):
  def body(x_ref, w_ref, o_ref):
    o_ref[...] = lax.dot_general(x_ref[...], w_ref[...],
                                 (((1,), (0,)), ((), ())),
                                 preferred_element_type=jnp.float32,
                                 precision=lax.Precision.HIGHEST)

  K, Dout = W.shape
  return pl.pallas_call(
      body,
      grid=(_N // _RB,),
      in_specs=[pl.BlockSpec((_RB, K), lambda i: (i, 0)),
                pl.BlockSpec((K, Dout), lambda i: (0, 0))],
      out_specs=pl.BlockSpec((_RB, Dout), lambda i: (i, 0)),
      out_shape=jax.ShapeDtypeStruct((_N, Dout), jnp.float32),
  )(x, W)


def _g_scale(dacc, h0):
  def body(d_ref, h_ref, g_ref, o_ref):
    deg = d_ref[0, :, :1] + d_ref[1, :, :1] + 1.0
    g = jnp.broadcast_to(lax.rsqrt(deg), (_RB, 2 * _DH))
    g_ref[...] = g
    h = h_ref[...] * g
    o_ref[0] = h[:, :_DH]
    o_ref[1] = h[:, _DH:]

  return pl.pallas_call(
      body,
      grid=(_N // _RB,),
      in_specs=[pl.BlockSpec((_NC, _RB, 16), lambda i: (0, i, 0)),
                pl.BlockSpec((_RB, 2 * _DH), lambda i: (i, 0))],
      out_specs=[pl.BlockSpec((_RB, 2 * _DH), lambda i: (i, 0)),
                 pl.BlockSpec((_NC, _RB, _DH), lambda i: (0, i, 0))],
      out_shape=[jax.ShapeDtypeStruct((_N, 2 * _DH), jnp.float32),
                 jax.ShapeDtypeStruct((_NC, _N, _DH), jnp.float32)],
  )(dacc, h0)


def _mid(acc, hpp3, g, b, W, split_out):
  def body(a_ref, h_ref, g_ref, b_ref, w_ref, o_ref):
    s0 = a_ref[0] + h_ref[0]
    s1 = a_ref[1] + h_ref[1]
    s = jnp.concatenate([s0, s1], axis=1)
    g = g_ref[...]
    t = jnp.maximum(g * s + b_ref[...], 0.0)
    h = lax.dot_general(t, w_ref[...], (((1,), (0,)), ((), ())),
                        preferred_element_type=jnp.float32,
                        precision=lax.Precision.HIGHEST)
    h = h * g[:, :h.shape[1]] if not split_out else h * g
    if split_out:
      o_ref[0] = h[:, :_DH]
      o_ref[1] = h[:, _DH:]
    else:
      o_ref[...] = h

  K, Dout = W.shape
  if split_out:
    out_spec = pl.BlockSpec((_NC, _RB, _DH), lambda i: (0, i, 0))
    out_shape = jax.ShapeDtypeStruct((_NC, _N, _DH), jnp.float32)
  else:
    out_spec = pl.BlockSpec((_RB, Dout), lambda i: (i, 0))
    out_shape = jax.ShapeDtypeStruct((_N, Dout), jnp.float32)
  return pl.pallas_call(
      body,
      grid=(_N // _RB,),
      in_specs=[pl.BlockSpec((_NC, _RB, _DH), lambda i: (0, i, 0)),
                pl.BlockSpec((_NC, _RB, _DH), lambda i: (0, i, 0)),
                pl.BlockSpec((_RB, 2 * _DH), lambda i: (i, 0)),
                pl.BlockSpec((1, K), lambda i: (0, 0)),
                pl.BlockSpec((K, Dout), lambda i: (0, 0))],
      out_specs=out_spec,
      out_shape=out_shape,
  )(acc, hpp3, g, b, W)


def _final(acc, hpp, g, b):
  def body(a_ref, h_ref, g_ref, b_ref, o_ref):
    g = g_ref[...][:, :h_ref.shape[1]]
    o_ref[...] = g * (a_ref[0] + a_ref[1] + h_ref[...]) + b_ref[...]

  D = hpp.shape[1]
  return pl.pallas_call(
      body,
      grid=(_N // _RB,),
      in_specs=[pl.BlockSpec((_NC, _RB, D), lambda i: (0, i, 0)),
                pl.BlockSpec((_RB, D), lambda i: (i, 0)),
                pl.BlockSpec((_RB, 2 * _DH), lambda i: (i, 0)),
                pl.BlockSpec((1, D), lambda i: (0, 0))],
      out_specs=pl.BlockSpec((_RB, D), lambda i: (i, 0)),
      out_shape=jax.ShapeDtypeStruct((_N, D), jnp.float32),
  )(acc, hpp, g, b)


def kernel(x, edge_index, W0, b0, W1, b1, W2, b2, W3, b3, W4, b4):
  row_w = edge_index[0].reshape(_NW, _KJW, _CH)
  col_w = edge_index[1].reshape(_NW, _KJW, _CH)
  zeros64 = jnp.zeros((_RPT, _DH), jnp.float32)
  zeros16 = jnp.zeros((_RPT, 16), jnp.float32)
  ones16 = jnp.ones((_CH, 16), jnp.float32)

  dacc = _deg_hist(col_w, ones16, zeros16)
  h0 = _mm_plain(x, W0)
  g, hpp3 = _g_scale(dacc, h0)

  W4p = jnp.pad(W4, ((0, 0), (0, 14)))
  b4p = jnp.pad(b4, (0, 14)).reshape(1, 16)

  bs = (b0.reshape(1, -1), b1.reshape(1, -1), b2.reshape(1, -1),
        b3.reshape(1, -1))
  Ws = (W1, W2, W3, W4p)
  for i in range(4):
    acc = _seg_sum_split(hpp3, row_w, col_w, zeros64)
    hpp3 = _mid(acc, hpp3, g, bs[i], Ws[i], split_out=(i < 3))
  acc = _seg_sum_part(hpp3, row_w, col_w, zeros16, 16)
  out16 = _final(acc, hpp3, g, b4p)
  return out16[:, :2]


# speedup vs baseline: 27.3360x; 1.1371x over previous
import jax
import jax.numpy as jnp
from jax import lax
from jax.experimental import pallas as pl
from jax.experimental.pallas import tpu as pltpu
from jax.experimental.pallas import tpu_sc as plsc

_N = 10000
_E = 320000
_NC = 2
_NS = 16
_NW = _NC * _NS
_CH = 80
_KJS = _E // _NS // _CH
_KJW = _E // _NW // _CH
_NP = 10240
_RPT = _NP // _NS
_DH = 64
_NB = 8
_RB = 2000

_mesh = plsc.VectorSubcoreMesh(core_axis_name="c", subcore_axis_name="s")
_sc_params = pltpu.CompilerParams(use_tc_tiling_on_sc=False)


def _ring_pipeline(tbl, row_v, col_v, acc_sh, msgs, gs, ss, kj):
  def wait_gather(jj, b):
    pltpu.make_async_copy(tbl.at[row_v.at[jj]], msgs[b], gs[b]).wait()

  def start_scatter(jj, b):
    pltpu.async_copy(msgs[b], acc_sh.at[col_v.at[jj]], ss[b], add=True)

  def drain_scatter(b):
    pltpu.make_async_copy(msgs[b], acc_sh.at[col_v.at[0]], ss[b]).wait()

  for jj in range(4):
    pltpu.async_copy(tbl.at[row_v.at[jj]], msgs[jj], gs[jj])
  for jj in range(4):
    wait_gather(jj, jj)
    start_scatter(jj, jj)
    pltpu.async_copy(tbl.at[row_v.at[jj + 4]], msgs[jj + 4], gs[jj + 4])

  main_hi = 4 + 8 * ((kj - 8) // 8)

  @pl.loop(4, main_hi, step=8)
  def _(jj0):
    for u in range(8):
      jj = jj0 + u
      b = (4 + u) % _NB
      wait_gather(jj, b)
      start_scatter(jj, b)
      b4 = (4 + u + 4) % _NB
      drain_scatter(b4)
      pltpu.async_copy(tbl.at[row_v.at[jj + 4]], msgs[b4], gs[b4])

  for jj in range(main_hi, kj):
    b = jj % _NB
    wait_gather(jj, b)
    start_scatter(jj, b)
    if jj + 4 < kj:
      b4 = (jj + 4) % _NB
      drain_scatter(b4)
      pltpu.async_copy(tbl.at[row_v.at[jj + 4]], msgs[b4], gs[b4])
  for b in range(_NB):
    drain_scatter(b)


def _sc_scratch(kj, d):
  return ([pltpu.VMEM((kj, _CH), jnp.int32),
           pltpu.VMEM((kj, _CH), jnp.int32)]
          + [pltpu.VMEM((_CH, d), jnp.float32) for _ in range(_NB)]
          + [pltpu.SemaphoreType.DMA for _ in range(2 * _NB)])


def _seg_sum_split(hpp, row2, col2, zeros):
  @pl.kernel(
      out_type=jax.ShapeDtypeStruct((_NP, 2 * _DH), jnp.float32),
      mesh=_mesh,
      compiler_params=_sc_params,
      scratch_types=_sc_scratch(_KJS, _DH)
      + [pltpu.VMEM_SHARED((_NP, _DH), jnp.float32)],
  )
  def k(hpp_hbm, row_hbm, col_hbm, zeros_hbm, out_hbm,
        row_v, col_v, *rest):
    msgs, gs, ss, acc_sh = (rest[:_NB], rest[_NB:2 * _NB],
                            rest[2 * _NB:3 * _NB], rest[3 * _NB])
    cid = lax.axis_index("c")
    sid = lax.axis_index("s")
    tbl = hpp_hbm
    pltpu.sync_copy(zeros_hbm, acc_sh.at[pl.ds(sid * _RPT, _RPT)])
    pltpu.sync_copy(row_hbm.at[cid, 2 * sid], row_v.at[pl.ds(0, _KJW)])
    pltpu.sync_copy(row_hbm.at[cid, 2 * sid + 1],
                    row_v.at[pl.ds(_KJW, _KJW)])
    pltpu.sync_copy(col_hbm.at[2 * sid], col_v.at[pl.ds(0, _KJW)])
    pltpu.sync_copy(col_hbm.at[2 * sid + 1], col_v.at[pl.ds(_KJW, _KJW)])
    plsc.subcore_barrier()
    _ring_pipeline(tbl, row_v, col_v, acc_sh, msgs, gs, ss, _KJS)
    plsc.subcore_barrier()
    pltpu.sync_copy(acc_sh.at[pl.ds(sid * _RPT, _RPT)],
                    out_hbm.at[pl.ds(sid * _RPT, _RPT),
                               pl.ds(cid * _DH, _DH)])

  return k(hpp, row2, col2, zeros)


def _seg_sum_part(hpp, row2, col2, zeros, D):
  @pl.kernel(
      out_type=jax.ShapeDtypeStruct((_NC, _NP, D), jnp.float32),
      mesh=_mesh,
      compiler_params=_sc_params,
      scratch_types=_sc_scratch(_KJW, D)
      + [pltpu.VMEM_SHARED((_NP, D), jnp.float32)],
  )
  def k(hpp_hbm, row_hbm, col_hbm, zeros_hbm, out_hbm,
        row_v, col_v, *rest):
    msgs, gs, ss, acc_sh = (rest[:_NB], rest[_NB:2 * _NB],
                            rest[2 * _NB:3 * _NB], rest[3 * _NB])
    cid = lax.axis_index("c")
    sid = lax.axis_index("s")
    wid = sid * _NC + cid
    pltpu.sync_copy(zeros_hbm, acc_sh.at[pl.ds(sid * _RPT, _RPT)])
    pltpu.sync_copy(row_hbm.at[wid], row_v)
    pltpu.sync_copy(col_hbm.at[wid], col_v)
    plsc.subcore_barrier()
    _ring_pipeline(hpp_hbm, row_v, col_v, acc_sh, msgs, gs, ss, _KJW)
    plsc.subcore_barrier()
    pltpu.sync_copy(acc_sh.at[pl.ds(sid * _RPT, _RPT)],
                    out_hbm.at[cid, pl.ds(sid * _RPT, _RPT)])

  return k(hpp, row2, col2, zeros)


def _deg_hist(col2, ones, zeros):
  @pl.kernel(
      out_type=jax.ShapeDtypeStruct((_NC, _NP, 16), jnp.float32),
      mesh=_mesh,
      compiler_params=_sc_params,
      scratch_types=[
          pltpu.VMEM((_KJW, _CH), jnp.int32),
          pltpu.VMEM((_CH, 16), jnp.float32),
          pltpu.SemaphoreType.DMA,
          pltpu.VMEM_SHARED((_NP, 16), jnp.float32),
      ],
  )
  def k(col_hbm, ones_hbm, zeros_hbm, out_hbm, col_v, ones_v, sem, acc_sh):
    cid = lax.axis_index("c")
    sid = lax.axis_index("s")
    wid = sid * _NC + cid
    pltpu.sync_copy(zeros_hbm, acc_sh.at[pl.ds(sid * _RPT, _RPT)])
    pltpu.sync_copy(col_hbm.at[wid], col_v)
    pltpu.sync_copy(ones_hbm, ones_v)
    plsc.subcore_barrier()

    @pl.loop(0, _KJW)
    def _(j):
      pltpu.async_copy(ones_v, acc_sh.at[col_v.at[j]], sem, add=True)

    @pl.loop(0, _KJW)
    def _(j):
      pltpu.make_async_copy(ones_v, acc_sh.at[col_v.at[0]], sem).wait()

    plsc.subcore_barrier()
    pltpu.sync_copy(acc_sh.at[pl.ds(sid * _RPT, _RPT)],
                    out_hbm.at[cid, pl.ds(sid * _RPT, _RPT)])

  return k(col2, ones, zeros)


def _mm_plain(x, W):
  def body(x_ref, w_ref, o_ref):
    o_ref[...] = lax.dot_general(x_ref[...], w_ref[...],
                                 (((1,), (0,)), ((), ())),
                                 preferred_element_type=jnp.float32,
                                 precision=lax.Precision.HIGHEST)

  K, Dout = W.shape
  return pl.pallas_call(
      body,
      grid=(_N // _RB,),
      in_specs=[pl.BlockSpec((_RB, K), lambda i: (i, 0)),
                pl.BlockSpec((K, Dout), lambda i: (0, 0))],
      out_specs=pl.BlockSpec((_RB, Dout), lambda i: (i, 0)),
      out_shape=jax.ShapeDtypeStruct((_N, Dout), jnp.float32),
  )(x, W)


def _g_scale(dacc, h0):
  def body(d_ref, h_ref, g_ref, o_ref):
    deg = d_ref[0, :, :1] + d_ref[1, :, :1] + 1.0
    g = jnp.broadcast_to(lax.rsqrt(deg), (_RB, 2 * _DH))
    g_ref[...] = g
    o_ref[...] = h_ref[...] * g

  return pl.pallas_call(
      body,
      grid=(_N // _RB,),
      in_specs=[pl.BlockSpec((_NC, _RB, 16), lambda i: (0, i, 0)),
                pl.BlockSpec((_RB, 2 * _DH), lambda i: (i, 0))],
      out_specs=[pl.BlockSpec((_RB, 2 * _DH), lambda i: (i, 0)),
                 pl.BlockSpec((_RB, 2 * _DH), lambda i: (i, 0))],
      out_shape=[jax.ShapeDtypeStruct((_N, 2 * _DH), jnp.float32),
                 jax.ShapeDtypeStruct((_N, 2 * _DH), jnp.float32)],
  )(dacc, h0)


def _mid(acc, hpp, g, b, W):
  def body(a_ref, h_ref, g_ref, b_ref, w_ref, o_ref):
    g = g_ref[...]
    t = jnp.maximum(g * (a_ref[...] + h_ref[...]) + b_ref[...], 0.0)
    h = lax.dot_general(t, w_ref[...], (((1,), (0,)), ((), ())),
                        preferred_element_type=jnp.float32,
                        precision=lax.Precision.HIGHEST)
    o_ref[...] = h * g[:, :h.shape[1]]

  K, Dout = W.shape
  return pl.pallas_call(
      body,
      grid=(_N // _RB,),
      in_specs=[pl.BlockSpec((_RB, K), lambda i: (i, 0)),
                pl.BlockSpec((_RB, K), lambda i: (i, 0)),
                pl.BlockSpec((_RB, 2 * _DH), lambda i: (i, 0)),
                pl.BlockSpec((1, K), lambda i: (0, 0)),
                pl.BlockSpec((K, Dout), lambda i: (0, 0))],
      out_specs=pl.BlockSpec((_RB, Dout), lambda i: (i, 0)),
      out_shape=jax.ShapeDtypeStruct((_N, Dout), jnp.float32),
  )(acc, hpp, g, b, W)


def _final(acc, hpp, g, b):
  def body(a_ref, h_ref, g_ref, b_ref, o_ref):
    g = g_ref[...][:, :h_ref.shape[1]]
    o_ref[...] = g * (a_ref[0] + a_ref[1] + h_ref[...]) + b_ref[...]

  D = hpp.shape[1]
  return pl.pallas_call(
      body,
      grid=(_N // _RB,),
      in_specs=[pl.BlockSpec((_NC, _RB, D), lambda i: (0, i, 0)),
                pl.BlockSpec((_RB, D), lambda i: (i, 0)),
                pl.BlockSpec((_RB, 2 * _DH), lambda i: (i, 0)),
                pl.BlockSpec((1, D), lambda i: (0, 0))],
      out_specs=pl.BlockSpec((_RB, D), lambda i: (i, 0)),
      out_shape=jax.ShapeDtypeStruct((_N, D), jnp.float32),
  )(acc, hpp, g, b)


def kernel(x, edge_index, W0, b0, W1, b1, W2, b2, W3, b3, W4, b4):
  row_w = edge_index[0].reshape(_NW, _KJW, _CH)
  col_w = edge_index[1].reshape(_NW, _KJW, _CH)
  zeros64 = jnp.zeros((_RPT, _DH), jnp.float32)
  zeros16 = jnp.zeros((_RPT, 16), jnp.float32)
  ones16 = jnp.ones((_CH, 16), jnp.float32)

  dacc = _deg_hist(col_w, ones16, zeros16)
  h0 = _mm_plain(x, W0)
  g, hpp = _g_scale(dacc, h0)

  W4p = jnp.pad(W4, ((0, 0), (0, 14)))
  b4p = jnp.pad(b4, (0, 14)).reshape(1, 16)

  bs = (b0.reshape(1, -1), b1.reshape(1, -1), b2.reshape(1, -1),
        b3.reshape(1, -1))
  Ws = (W1, W2, W3, W4p)
  rowx2 = jnp.stack([2 * edge_index[0], 2 * edge_index[0] + 1])
  rowx2 = rowx2.reshape(_NC, _NW, _KJW, _CH)
  for i in range(4):
    acc = _seg_sum_split(hpp.reshape(2 * _N, _DH), rowx2, col_w, zeros64)
    hpp = _mid(acc, hpp, g, bs[i], Ws[i])
  acc = _seg_sum_part(hpp, row_w, col_w, zeros16, 16)
  out16 = _final(acc, hpp, g, b4p)
  return out16[:, :2]


# speedup vs baseline: 28.1252x; 1.0289x over previous
import jax
import jax.numpy as jnp
from jax import lax
from jax.experimental import pallas as pl
from jax.experimental.pallas import tpu as pltpu
from jax.experimental.pallas import tpu_sc as plsc

_N = 10000
_E = 320000
_NC = 2
_NS = 16
_NW = _NC * _NS
_CH = 80
_KJS = _E // _NS // _CH
_KJW = _E // _NW // _CH
_NP = 10240
_RPT = _NP // _NS
_DH = 64
_NB = 8
_RB = 2000

_mesh = plsc.VectorSubcoreMesh(core_axis_name="c", subcore_axis_name="s")
_sc_params = pltpu.CompilerParams(use_tc_tiling_on_sc=False)


def _ring_pipeline(tbl, row_v, col_v, acc_sh, msgs, gs, ss, kj):
  def idx(ref, jj):
    return ref.at[pl.ds(jj * _CH, _CH)]

  def wait_gather(jj, b):
    pltpu.make_async_copy(tbl.at[idx(row_v, jj)], msgs[b], gs[b]).wait()

  def start_scatter(jj, b):
    pltpu.async_copy(msgs[b], acc_sh.at[idx(col_v, jj)], ss[b], add=True)

  def drain_scatter(b):
    pltpu.make_async_copy(msgs[b], acc_sh.at[idx(col_v, 0)], ss[b]).wait()

  for jj in range(4):
    pltpu.async_copy(tbl.at[idx(row_v, jj)], msgs[jj], gs[jj])
  for jj in range(4):
    wait_gather(jj, jj)
    start_scatter(jj, jj)
    pltpu.async_copy(tbl.at[idx(row_v, jj + 4)], msgs[jj + 4], gs[jj + 4])

  main_hi = 4 + 8 * ((kj - 8) // 8)

  @pl.loop(4, main_hi, step=8)
  def _(jj0):
    for u in range(8):
      jj = jj0 + u
      b = (4 + u) % _NB
      wait_gather(jj, b)
      start_scatter(jj, b)
      b4 = (4 + u + 4) % _NB
      drain_scatter(b4)
      pltpu.async_copy(tbl.at[idx(row_v, jj + 4)], msgs[b4], gs[b4])

  for jj in range(main_hi, kj):
    b = jj % _NB
    wait_gather(jj, b)
    start_scatter(jj, b)
    if jj + 4 < kj:
      b4 = (jj + 4) % _NB
      drain_scatter(b4)
      pltpu.async_copy(tbl.at[idx(row_v, jj + 4)], msgs[b4], gs[b4])
  for b in range(_NB):
    drain_scatter(b)


def _sc_scratch(kj, d):
  return ([pltpu.VMEM((kj * _CH,), jnp.int32),
           pltpu.VMEM((kj * _CH,), jnp.int32)]
          + [pltpu.VMEM((_CH, d), jnp.float32) for _ in range(_NB)]
          + [pltpu.SemaphoreType.DMA for _ in range(2 * _NB)])


def _seg_sum_split(hpp, row2, col2, zeros):
  @pl.kernel(
      out_type=jax.ShapeDtypeStruct((_NP, 2 * _DH), jnp.float32),
      mesh=_mesh,
      compiler_params=_sc_params,
      scratch_types=_sc_scratch(_KJS, _DH)
      + [pltpu.VMEM_SHARED((_NP, _DH), jnp.float32)],
  )
  def k(hpp_hbm, row_hbm, col_hbm, zeros_hbm, out_hbm,
        row_v, col_v, *rest):
    msgs, gs, ss, acc_sh = (rest[:_NB], rest[_NB:2 * _NB],
                            rest[2 * _NB:3 * _NB], rest[3 * _NB])
    cid = lax.axis_index("c")
    sid = lax.axis_index("s")
    tbl = hpp_hbm
    epw = _KJS * _CH
    pltpu.sync_copy(zeros_hbm, acc_sh.at[pl.ds(sid * _RPT, _RPT)])
    pltpu.sync_copy(row_hbm.at[pl.ds(cid * _E + sid * epw, epw)], row_v)
    pltpu.sync_copy(col_hbm.at[pl.ds(sid * epw, epw)], col_v)
    plsc.subcore_barrier()
    _ring_pipeline(tbl, row_v, col_v, acc_sh, msgs, gs, ss, _KJS)
    plsc.subcore_barrier()
    pltpu.sync_copy(acc_sh.at[pl.ds(sid * _RPT, _RPT)],
                    out_hbm.at[pl.ds(sid * _RPT, _RPT),
                               pl.ds(cid * _DH, _DH)])

  return k(hpp, row2, col2, zeros)


def _seg_sum_part(hpp, row2, col2, zeros, D):
  @pl.kernel(
      out_type=jax.ShapeDtypeStruct((_NC, _NP, D), jnp.float32),
      mesh=_mesh,
      compiler_params=_sc_params,
      scratch_types=_sc_scratch(_KJW, D)
      + [pltpu.VMEM_SHARED((_NP, D), jnp.float32)],
  )
  def k(hpp_hbm, row_hbm, col_hbm, zeros_hbm, out_hbm,
        row_v, col_v, *rest):
    msgs, gs, ss, acc_sh = (rest[:_NB], rest[_NB:2 * _NB],
                            rest[2 * _NB:3 * _NB], rest[3 * _NB])
    cid = lax.axis_index("c")
    sid = lax.axis_index("s")
    wid = sid * _NC + cid
    epw = _KJW * _CH
    pltpu.sync_copy(zeros_hbm, acc_sh.at[pl.ds(sid * _RPT, _RPT)])
    pltpu.sync_copy(row_hbm.at[pl.ds(wid * epw, epw)], row_v)
    pltpu.sync_copy(col_hbm.at[pl.ds(wid * epw, epw)], col_v)
    plsc.subcore_barrier()
    _ring_pipeline(hpp_hbm, row_v, col_v, acc_sh, msgs, gs, ss, _KJW)
    plsc.subcore_barrier()
    pltpu.sync_copy(acc_sh.at[pl.ds(sid * _RPT, _RPT)],
                    out_hbm.at[cid, pl.ds(sid * _RPT, _RPT)])

  return k(hpp, row2, col2, zeros)


def _deg_hist(col2, ones, zeros):
  @pl.kernel(
      out_type=jax.ShapeDtypeStruct((_NP, 2 * _DH), jnp.float32),
      mesh=_mesh,
      compiler_params=_sc_params,
      scratch_types=[
          pltpu.VMEM((_KJW * _CH,), jnp.int32),
          pltpu.VMEM((_CH, 16), jnp.float32),
          pltpu.SemaphoreType.DMA,
          pltpu.VMEM_SHARED((_NP, 16), jnp.float32),
      ],
  )
  def k(col_hbm, ones_hbm, zeros_hbm, out_hbm, col_v, ones_v, sem, acc_sh):
    cid = lax.axis_index("c")
    sid = lax.axis_index("s")
    wid = sid * _NC + cid
    epw = _KJW * _CH
    pltpu.sync_copy(zeros_hbm, acc_sh.at[pl.ds(sid * _RPT, _RPT)])
    pltpu.sync_copy(col_hbm.at[pl.ds(wid * epw, epw)], col_v)
    pltpu.sync_copy(ones_hbm, ones_v)
    plsc.subcore_barrier()

    @pl.loop(0, _KJW)
    def _(j):
      pltpu.async_copy(ones_v, acc_sh.at[col_v.at[pl.ds(j * _CH, _CH)]],
                       sem, add=True)

    @pl.loop(0, _KJW)
    def _(j):
      pltpu.make_async_copy(ones_v, acc_sh.at[col_v.at[pl.ds(0, _CH)]], sem).wait()

    plsc.subcore_barrier()
    pltpu.sync_copy(acc_sh.at[pl.ds(sid * _RPT, _RPT)],
                    out_hbm.at[pl.ds(sid * _RPT, _RPT),
                               pl.ds(16 * cid, 16)])

  return k(col2, ones, zeros)


def _mm_plain(x, W):
  def body(x_ref, w_ref, o_ref):
    o_ref[...] = lax.dot_general(x_ref[...], w_ref[...],
                                 (((1,), (0,)), ((), ())),
                                 preferred_element_type=jnp.float32,
                                 precision=lax.Precision.HIGHEST)

  K, Dout = W.shape
  return pl.pallas_call(
      body,
      grid=(_N // _RB,),
      in_specs=[pl.BlockSpec((_RB, K), lambda i: (i, 0)),
                pl.BlockSpec((K, Dout), lambda i: (0, 0))],
      out_specs=pl.BlockSpec((_RB, Dout), lambda i: (i, 0)),
      out_shape=jax.ShapeDtypeStruct((_N, Dout), jnp.float32),
  )(x, W)


def _g_scale(dacc, h0):
  def body(d_ref, h_ref, g_ref, o_ref):
    deg = d_ref[:, :1] + d_ref[:, 16:17] + 1.0
    g = jnp.broadcast_to(lax.rsqrt(deg), (_RB, 2 * _DH))
    g_ref[...] = g
    o_ref[...] = h_ref[...] * g

  return pl.pallas_call(
      body,
      grid=(_N // _RB,),
      in_specs=[pl.BlockSpec((_RB, 2 * _DH), lambda i: (i, 0)),
                pl.BlockSpec((_RB, 2 * _DH), lambda i: (i, 0))],
      out_specs=[pl.BlockSpec((_RB, 2 * _DH), lambda i: (i, 0)),
                 pl.BlockSpec((_RB, 2 * _DH), lambda i: (i, 0))],
      out_shape=[jax.ShapeDtypeStruct((_N, 2 * _DH), jnp.float32),
                 jax.ShapeDtypeStruct((_N, 2 * _DH), jnp.float32)],
  )(dacc, h0)


def _mid(acc, hpp, g, b, W):
  def body(a_ref, h_ref, g_ref, b_ref, w_ref, o_ref):
    g = g_ref[...]
    t = jnp.maximum(g * (a_ref[...] + h_ref[...]) + b_ref[...], 0.0)
    h = lax.dot_general(t, w_ref[...], (((1,), (0,)), ((), ())),
                        preferred_element_type=jnp.float32,
                        precision=lax.Precision.HIGHEST)
    o_ref[...] = h * g[:, :h.shape[1]]

  K, Dout = W.shape
  return pl.pallas_call(
      body,
      grid=(_N // _RB,),
      in_specs=[pl.BlockSpec((_RB, K), lambda i: (i, 0)),
                pl.BlockSpec((_RB, K), lambda i: (i, 0)),
                pl.BlockSpec((_RB, 2 * _DH), lambda i: (i, 0)),
                pl.BlockSpec((1, K), lambda i: (0, 0)),
                pl.BlockSpec((K, Dout), lambda i: (0, 0))],
      out_specs=pl.BlockSpec((_RB, Dout), lambda i: (i, 0)),
      out_shape=jax.ShapeDtypeStruct((_N, Dout), jnp.float32),
  )(acc, hpp, g, b, W)


def _final(acc, hpp, g, b):
  def body(a_ref, h_ref, g_ref, b_ref, o_ref):
    g = g_ref[...][:, :h_ref.shape[1]]
    o_ref[...] = g * (a_ref[0] + a_ref[1] + h_ref[...]) + b_ref[...]

  D = hpp.shape[1]
  return pl.pallas_call(
      body,
      grid=(_N // _RB,),
      in_specs=[pl.BlockSpec((_NC, _RB, D), lambda i: (0, i, 0)),
                pl.BlockSpec((_RB, D), lambda i: (i, 0)),
                pl.BlockSpec((_RB, 2 * _DH), lambda i: (i, 0)),
                pl.BlockSpec((1, D), lambda i: (0, 0))],
      out_specs=pl.BlockSpec((_RB, D), lambda i: (i, 0)),
      out_shape=jax.ShapeDtypeStruct((_N, D), jnp.float32),
  )(acc, hpp, g, b)


def kernel(x, edge_index, W0, b0, W1, b1, W2, b2, W3, b3, W4, b4):
  row_fl = edge_index[0]
  col_fl = edge_index[1]
  zeros64 = jnp.zeros((_RPT, _DH), jnp.float32)
  zeros16 = jnp.zeros((_RPT, 16), jnp.float32)
  ones16 = jnp.ones((_CH, 16), jnp.float32)

  dacc = _deg_hist(col_fl, ones16, zeros16)
  h0 = _mm_plain(x, W0)
  g, hpp = _g_scale(dacc, h0)

  W4p = jnp.pad(W4, ((0, 0), (0, 14)))
  b4p = jnp.pad(b4, (0, 14)).reshape(1, 16)

  bs = (b0.reshape(1, -1), b1.reshape(1, -1), b2.reshape(1, -1),
        b3.reshape(1, -1))
  Ws = (W1, W2, W3, W4p)
  rowx2 = jnp.concatenate([2 * row_fl, 2 * row_fl + 1])
  for i in range(4):
    acc = _seg_sum_split(hpp.reshape(2 * _N, _DH), rowx2, col_fl, zeros64)
    hpp = _mid(acc, hpp, g, bs[i], Ws[i])
  acc = _seg_sum_part(hpp, row_fl, col_fl, zeros16, 16)
  out16 = _final(acc, hpp, g, b4p)
  return out16[:, :2]
